# baseline (speedup 1.0000x reference)
import numpy as np
import jax
import jax.numpy as jnp
from jax.experimental import pallas as pl

N = 10000
E = 160000
A = 128
H = 256
L = 4
G = 64
T = 1000


def _lin(p, name, x):
    return x @ p[name + '_w'] + p[name + '_b']


def _timestep_embedding(timesteps, dim):
    half = dim // 2
    freqs = jnp.exp(-np.log(10000.0) * jnp.arange(half, dtype=jnp.float32) / (half - 1))
    args = timesteps.astype(jnp.float32)[:, None] * freqs[None, :]
    return jnp.concatenate([jnp.sin(args), jnp.cos(args)], axis=1)


def _identity_pallas(x):
    def body(x_ref, o_ref):
        o_ref[...] = x_ref[...]
    blk = (2000, x.shape[1])
    return pl.pallas_call(
        body,
        grid=(x.shape[0] // blk[0],),
        in_specs=[pl.BlockSpec(blk, lambda i: (i, 0))],
        out_specs=pl.BlockSpec(blk, lambda i: (i, 0)),
        out_shape=jax.ShapeDtypeStruct(x.shape, x.dtype))(x)


def kernel(atom_type, pos, bond_index, batch, time_step, edge_type, edge_index, edge_length, params):
    relu = jax.nn.relu
    silu = jax.nn.silu
    temb = _timestep_embedding(time_step, H)
    temb = _lin(params, 'temb0', temb)
    temb = relu(temb)
    temb = _lin(params, 'temb1', temb)
    temb = _lin(params, 'temb_proj', relu(temb))
    h_in = jnp.concatenate([atom_type, temb[batch]], axis=1)
    row, col = edge_index[0], edge_index[1]
    edge2graph = batch[row]
    ea = relu(_lin(params, 'ee1', edge_length))
    ea = _lin(params, 'ee2', ea)
    edge_attr = ea * params['bond_emb'][edge_type]
    edge_attr = edge_attr + temb[edge2graph]
    h = _lin(params, 'emb_in', h_in)
    p = pos
    for l in range(L):
        rel = p[row] - p[col]
        d2 = jnp.sum(rel * rel, axis=-1, keepdims=True)
        rel_n = rel / (jnp.sqrt(d2) + 1e-8)
        m_in = jnp.concatenate([h[row], h[col], d2, edge_attr], axis=-1)
        m = silu(m_in @ params[f'e{l}_1_w'] + params[f'e{l}_1_b'])
        m = silu(m @ params[f'e{l}_2_w'] + params[f'e{l}_2_b'])
        gate = jax.nn.sigmoid(m @ params[f'e{l}_soft_w'] + params[f'e{l}_soft_b'])
        m = m * gate
        cw = silu(m @ params[f'c{l}_1_w'] + params[f'c{l}_1_b'])
        cw = cw @ params[f'c{l}_2_w'] + params[f'c{l}_2_b']
        p = p + jax.ops.segment_sum(rel_n * cw, row, num_segments=N)
        agg = jax.ops.segment_sum(m, row, num_segments=N)
        upd = silu(jnp.concatenate([h, agg], axis=-1) @ params[f'n{l}_1_w'] + params[f'n{l}_1_b'])
        upd = upd @ params[f'n{l}_2_w'] + params[f'n{l}_2_b']
        h = h + upd
    node_attr = h
    h_pair = jnp.concatenate([node_attr[row] * node_attr[col], edge_attr], axis=-1)
    d = relu(_lin(params, 'd1', h_pair))
    d = relu(_lin(params, 'd2', d))
    dist_score = _lin(params, 'd3', d)
    g = relu(_lin(params, 'g1', node_attr))
    g = relu(_lin(params, 'g2', g))
    node_score = _lin(params, 'g3', g)
    local_edge_mask = (edge_type == 0)
    dist_score = _identity_pallas(dist_score)
    return (dist_score, node_score, edge_index, edge_type, edge_length, local_edge_mask)


# speedup vs baseline: 1.0484x; 1.0484x over previous
import numpy as np
import jax
import jax.numpy as jnp
from jax.experimental import pallas as pl

N = 10000
E = 160000
A = 128
H = 256
L = 4
G = 64

TN = 1000
TE = 2000

_relu = jax.nn.relu
_sigmoid = jax.nn.sigmoid


def _silu(x):
    return x * _sigmoid(x)


def _mm(a, b):
    return jax.lax.dot_general(a, b, (((1,), (0,)), ((), ())),
                               preferred_element_type=jnp.float32)


def _rows(r, c):
    return pl.BlockSpec((r, c), lambda i: (i, 0))


def _fixed(shape):
    return pl.BlockSpec(shape, lambda i: (0, 0))


def _f32(*shape):
    return jax.ShapeDtypeStruct(shape, jnp.float32)


def _temb_body(ts_ref, w0, b0, w1, b1, w2, b2, out_ref):
    ts = ts_ref[...].astype(jnp.float32)
    j = jax.lax.broadcasted_iota(jnp.int32, (1, H // 2), 1).astype(jnp.float32)
    freqs = jnp.exp(-np.log(10000.0) * j / (H // 2 - 1))
    args = ts * freqs
    te = jnp.concatenate([jnp.sin(args), jnp.cos(args)], axis=1)
    x = _mm(te, w0[...]) + b0[...]
    x = _relu(x)
    x = _mm(x, w1[...]) + b1[...]
    x = _mm(_relu(x), w2[...]) + b2[...]
    out_ref[...] = x


def _temb(time_step, p):
    return pl.pallas_call(
        _temb_body,
        out_shape=_f32(G, H),
    )(time_step.reshape(G, 1),
      p['temb0_w'], p['temb0_b'].reshape(1, -1),
      p['temb1_w'], p['temb1_b'].reshape(1, -1),
      p['temb_proj_w'], p['temb_proj_b'].reshape(1, -1))


def _nodepro_body(at_ref, b_ref, temb_ref, ea_w, eb_w, e_b, h_ref, tn_ref):
    oh = (b_ref[...] == jax.lax.broadcasted_iota(jnp.int32, (TN, G), 1))
    tn = _mm(oh.astype(jnp.float32), temb_ref[...])
    h_ref[...] = _mm(at_ref[...], ea_w[...]) + _mm(tn, eb_w[...]) + e_b[...]
    tn_ref[...] = tn


def _node_prologue(atom_type, batch, temb, p):
    emb_a = p['emb_in_w'][:A]
    emb_b = p['emb_in_w'][A:]
    return pl.pallas_call(
        _nodepro_body,
        grid=(N // TN,),
        in_specs=[_rows(TN, A), _rows(TN, 1), _fixed((G, H)),
                  _fixed((A, H)), _fixed((H, H)), _fixed((1, H))],
        out_specs=[_rows(TN, H), _rows(TN, H)],
        out_shape=[_f32(N, H), _f32(N, H)],
    )(atom_type, batch.reshape(N, 1).astype(jnp.int32), temb,
      emb_a, emb_b, p['emb_in_b'].reshape(1, H))


def _edgepro_body(el_ref, et_ref, ee1w, ee1b, ee2w, ee2b, bond4, eab_ref):
    ea = _relu(el_ref[...] * ee1w[...] + ee1b[...])
    ea = _mm(ea, ee2w[...]) + ee2b[...]
    oh = (et_ref[...] == jax.lax.broadcasted_iota(jnp.int32, (TE, 4), 1))
    eab_ref[...] = ea * _mm(oh.astype(jnp.float32), bond4[...])


def _edge_prologue(edge_length, edge_type, p):
    return pl.pallas_call(
        _edgepro_body,
        grid=(E // TE,),
        in_specs=[_rows(TE, 1), _rows(TE, 1), _fixed((1, H)), _fixed((1, H)),
                  _fixed((H, H)), _fixed((1, H)), _fixed((4, H))],
        out_specs=_rows(TE, H),
        out_shape=_f32(E, H),
    )(edge_length, edge_type.reshape(E, 1).astype(jnp.int32),
      p['ee1_w'], p['ee1_b'].reshape(1, H),
      p['ee2_w'], p['ee2_b'].reshape(1, H), p['bond_emb'][:4])


def _nodeside_body(h_ref, tn_ref, wrow, wcol, wea, rt_ref, ct_ref):
    h = h_ref[...]
    rt_ref[...] = _mm(h, wrow[...]) + _mm(tn_ref[...], wea[...])
    ct_ref[...] = _mm(h, wcol[...])


def _nodeside(h, tn, wrow, wcol, wea):
    return pl.pallas_call(
        _nodeside_body,
        grid=(N // TN,),
        in_specs=[_rows(TN, H), _rows(TN, H),
                  _fixed((H, H)), _fixed((H, H)), _fixed((H, H))],
        out_specs=[_rows(TN, H), _rows(TN, H)],
        out_shape=[_f32(N, H), _f32(N, H)],
    )(h, tn, wrow, wcol, wea)


def _edge_body(grow, gcol, prow, pcol, eab, wea, wd2, b1, w2, b2,
               wsoft, bsoft, wc1, bc1, wc2, bc2, m_ref, rc_ref):
    rel = prow[...] - pcol[...]
    d2 = jnp.sum(rel * rel, axis=1, keepdims=True)
    reln = rel / (jnp.sqrt(d2) + 1e-8)
    m1 = _silu(grow[...] + gcol[...] + _mm(eab[...], wea[...])
               + d2 * wd2[...] + b1[...])
    m = _silu(_mm(m1, w2[...]) + b2[...])
    gate = _sigmoid(_mm(m, wsoft[...]) + bsoft[...])
    m = m * gate
    cw = _silu(_mm(m, wc1[...]) + bc1[...])
    cws = _mm(cw, wc2[...]) + bc2[...]
    m_ref[...] = m
    rc_ref[...] = reln * cws


def _edge_layer(grow, gcol, prow, pcol, eab, wea, wd2, b1, w2, b2,
                wsoft, bsoft, wc1, bc1, wc2, bc2):
    return pl.pallas_call(
        _edge_body,
        grid=(E // TE,),
        in_specs=[_rows(TE, H), _rows(TE, H), _rows(TE, 8), _rows(TE, 8),
                  _rows(TE, H),
                  _fixed((H, H)), _fixed((1, H)), _fixed((1, H)),
                  _fixed((H, H)), _fixed((1, H)),
                  _fixed((H, 1)), _fixed((1, 1)),
                  _fixed((H, H)), _fixed((1, H)),
                  _fixed((H, 1)), _fixed((1, 1))],
        out_specs=[_rows(TE, H), _rows(TE, 8)],
        out_shape=[_f32(E, H), _f32(E, 8)],
    )(grow, gcol, prow, pcol, eab, wea, wd2, b1, w2, b2,
      wsoft, bsoft, wc1, bc1, wc2, bc2)


def _nodeupd_body(h_ref, agg_ref, aggp_ref, p_ref, n1a, n1b, nb1, n2w, n2b,
                  ho_ref, po_ref):
    h = h_ref[...]
    upd = _silu(_mm(h, n1a[...]) + _mm(agg_ref[...], n1b[...]) + nb1[...])
    ho_ref[...] = h + _mm(upd, n2w[...]) + n2b[...]
    po_ref[...] = p_ref[...] + aggp_ref[...]


def _node_update(h, agg, aggp, pp, n1a, n1b, nb1, n2w, n2b):
    return pl.pallas_call(
        _nodeupd_body,
        grid=(N // TN,),
        in_specs=[_rows(TN, H), _rows(TN, H), _rows(TN, 8), _rows(TN, 8),
                  _fixed((H, H)), _fixed((H, H)), _fixed((1, H)),
                  _fixed((H, H)), _fixed((1, H))],
        out_specs=[_rows(TN, H), _rows(TN, 8)],
        out_shape=[_f32(N, H), _f32(N, 8)],
    )(h, agg, aggp, pp, n1a, n1b, nb1, n2w, n2b)


def _nodehead_body(h_ref, tn_ref, g1w, g1b, g2w, g2b, g3w, g3b, wde,
                   ns_ref, tf_ref):
    h = h_ref[...]
    g = _relu(_mm(h, g1w[...]) + g1b[...])
    g = _relu(_mm(g, g2w[...]) + g2b[...])
    ns_ref[...] = _mm(g, g3w[...]) + g3b[...]
    tf_ref[...] = _mm(tn_ref[...], wde[...])


def _node_head(h, tn, p, wde):
    return pl.pallas_call(
        _nodehead_body,
        grid=(N // TN,),
        in_specs=[_rows(TN, H), _rows(TN, H),
                  _fixed((H, H)), _fixed((1, H)),
                  _fixed((H, H // 2)), _fixed((1, H // 2)),
                  _fixed((H // 2, A)), _fixed((1, A)), _fixed((H, H))],
        out_specs=[_rows(TN, A), _rows(TN, H)],
        out_shape=[_f32(N, A), _f32(N, H)],
    )(h, tn, p['g1_w'], p['g1_b'].reshape(1, -1),
      p['g2_w'], p['g2_b'].reshape(1, -1),
      p['g3_w'], p['g3_b'].reshape(1, -1), wde)


def _disthead_body(hr, hc, tfr, eab, wdh, wde, d1b, d2w, d2b, d3w, d3b, out):
    hp = _relu(_mm(hr[...] * hc[...], wdh[...]) + _mm(eab[...], wde[...])
               + tfr[...] + d1b[...])
    d = _relu(_mm(hp, d2w[...]) + d2b[...])
    out[...] = _mm(d, d3w[...]) + d3b[...]


def _dist_head(hr, hc, tfr, eab, p):
    wdh = p['d1_w'][:H]
    wde = p['d1_w'][H:]
    return pl.pallas_call(
        _disthead_body,
        grid=(E // TE,),
        in_specs=[_rows(TE, H), _rows(TE, H), _rows(TE, H), _rows(TE, H),
                  _fixed((H, H)), _fixed((H, H)), _fixed((1, H)),
                  _fixed((H, H // 2)), _fixed((1, H // 2)),
                  _fixed((H // 2, 1)), _fixed((1, 1))],
        out_specs=_rows(TE, 1),
        out_shape=_f32(E, 1),
    )(hr, hc, tfr, eab, wdh, wde, p['d1_b'].reshape(1, -1),
      p['d2_w'], p['d2_b'].reshape(1, -1), p['d3_w'], p['d3_b'].reshape(1, -1))


def kernel(atom_type, pos, bond_index, batch, time_step, edge_type, edge_index, edge_length, params):
    p = params
    row = edge_index[0]
    col = edge_index[1]

    temb = _temb(time_step, p)
    h, tn = _node_prologue(atom_type, batch, temb, p)
    eab = _edge_prologue(edge_length, edge_type, p)

    pp = jnp.concatenate([pos, jnp.zeros((N, 5), jnp.float32)], axis=1)

    for l in range(L):
        w1 = p[f'e{l}_1_w']
        rt, ct = _nodeside(h, tn, w1[:H], w1[H:2 * H], w1[2 * H + 1:])
        grow = rt[row]
        gcol = ct[col]
        prow = pp[row]
        pcol = pp[col]
        m, rc = _edge_layer(
            grow, gcol, prow, pcol, eab,
            w1[2 * H + 1:], w1[2 * H:2 * H + 1], p[f'e{l}_1_b'].reshape(1, H),
            p[f'e{l}_2_w'], p[f'e{l}_2_b'].reshape(1, H),
            p[f'e{l}_soft_w'], p[f'e{l}_soft_b'].reshape(1, 1),
            p[f'c{l}_1_w'], p[f'c{l}_1_b'].reshape(1, H),
            p[f'c{l}_2_w'], p[f'c{l}_2_b'].reshape(1, 1))
        agg = jax.ops.segment_sum(m, row, num_segments=N)
        aggp = jax.ops.segment_sum(rc, row, num_segments=N)
        n1w = p[f'n{l}_1_w']
        h, pp = _node_update(h, agg, aggp, pp,
                             n1w[:H], n1w[H:], p[f'n{l}_1_b'].reshape(1, H),
                             p[f'n{l}_2_w'], p[f'n{l}_2_b'].reshape(1, H))

    node_score, tf = _node_head(h, tn, p, p['d1_w'][H:])
    hr = h[row]
    hc = h[col]
    tfr = tf[row]
    dist_score = _dist_head(hr, hc, tfr, eab, p)

    local_edge_mask = (edge_type == 0)
    return (dist_score, node_score, edge_index, edge_type, edge_length, local_edge_mask)


# speedup vs baseline: 2.4768x; 2.3625x over previous
import numpy as np
import jax
import jax.numpy as jnp
from jax.experimental import pallas as pl
from jax.experimental.pallas import tpu as pltpu
from jax.experimental.pallas import tpu_sc as plsc

N = 10000
E = 160000
A = 128
H = 256
L = 4
G = 64

TN = 1000
TE = 2000

_relu = jax.nn.relu
_sigmoid = jax.nn.sigmoid


def _silu(x):
    return x * _sigmoid(x)


def _mm(a, b):
    return jax.lax.dot_general(a, b, (((1,), (0,)), ((), ())),
                               preferred_element_type=jnp.float32)


def _rows(r, c):
    return pl.BlockSpec((r, c), lambda i: (i, 0))


def _fixed(shape):
    return pl.BlockSpec(shape, lambda i: (0, 0))


def _f32(*shape):
    return jax.ShapeDtypeStruct(shape, jnp.float32)


_NS = 16
_CH = 128
_NCHUNK = E // _CH
_NPAD = 10240
_NROWS = _NPAD // _NS

_sc_cache = {}


def _sc_mesh():
    return plsc.VectorSubcoreMesh(core_axis_name="c", subcore_axis_name="s",
                                  num_cores=2, num_subcores=_NS)


def _chunk_loop(s, body):
    nk = (_NCHUNK - s + _NS - 1) // _NS

    def step(i, carry):
        base = (s + i * _NS) * _CH
        body(base)
        return carry

    jax.lax.fori_loop(0, nk, step, 0)


def _sc_gather_layer_body(rowtab, coltab, row, col,
                          grow, gcol,
                          idx_v, tbuf, sem_t):
    c = jax.lax.axis_index("c")
    s = jax.lax.axis_index("s")

    def run(tab, idxarr, outt):
        def one(base):
            pltpu.sync_copy(idxarr.at[pl.ds(base, _CH)], idx_v)
            pltpu.async_copy(tab.at[idx_v], tbuf, sem_t).wait()
            pltpu.sync_copy(tbuf, outt.at[pl.ds(base, _CH)])
        _chunk_loop(s, one)

    @pl.when(c == 0)
    def _():
        run(rowtab, row, grow)

    @pl.when(c == 1)
    def _():
        run(coltab, col, gcol)


def _sc_gather_layer(rt, ct, row, col):
    if 'gl' not in _sc_cache:
        _sc_cache['gl'] = pl.kernel(
            _sc_gather_layer_body,
            out_type=[_f32(E, H + 128), _f32(E, H + 128)],
            mesh=_sc_mesh(),
            scratch_types=[pltpu.VMEM((_CH,), jnp.int32),
                           pltpu.VMEM((_CH, H + 128), jnp.float32),
                           pltpu.SemaphoreType.DMA],
        )
    return _sc_cache['gl'](rt, ct, row, col)


def _sc_gather_final_body(tabf, htab, row, col, growf, gcolf,
                          idx_v, tbufw, tbufn, sem):
    c = jax.lax.axis_index("c")
    s = jax.lax.axis_index("s")

    @pl.when(c == 0)
    def _():
        def one(base):
            pltpu.sync_copy(row.at[pl.ds(base, _CH)], idx_v)
            pltpu.async_copy(tabf.at[idx_v], tbufw, sem).wait()
            pltpu.sync_copy(tbufw, growf.at[pl.ds(base, _CH)])
        _chunk_loop(s, one)

    @pl.when(c == 1)
    def _():
        def one(base):
            pltpu.sync_copy(col.at[pl.ds(base, _CH)], idx_v)
            pltpu.async_copy(htab.at[idx_v], tbufn, sem).wait()
            pltpu.sync_copy(tbufn, gcolf.at[pl.ds(base, _CH)])
        _chunk_loop(s, one)


def _sc_gather_final(tabf, htab, row, col):
    if 'gf' not in _sc_cache:
        _sc_cache['gf'] = pl.kernel(
            _sc_gather_final_body,
            out_type=[_f32(E, 2 * H), _f32(E, H)],
            mesh=_sc_mesh(),
            scratch_types=[pltpu.VMEM((_CH,), jnp.int32),
                           pltpu.VMEM((_CH, 2 * H), jnp.float32),
                           pltpu.VMEM((_CH, H), jnp.float32),
                           pltpu.SemaphoreType.DMA],
        )
    return _sc_cache['gf'](tabf, htab, row, col)


def _sc_scatter_body(m0, m1, rc, row, nid, z128,
                     agg0, agg1, aggp0, aggp1,
                     idx_v, mbuf, acc):
    c = jax.lax.axis_index("c")
    s = jax.lax.axis_index("s")

    def _rows_loop(body):
        def step(i, carry):
            body(s * _NROWS + i * _CH)
            return carry
        jax.lax.fori_loop(0, _NROWS // _CH, step, 0)

    def zero_acc():
        pltpu.sync_copy(z128.at[pl.ds(0, _CH)], mbuf)

        def z(base):
            pltpu.sync_copy(nid.at[pl.ds(base, _CH)], idx_v)
            pltpu.sync_copy(mbuf, acc.at[idx_v])
        _rows_loop(z)

    def madd(mref):
        def one(base):
            pltpu.sync_copy(row.at[pl.ds(base, _CH)], idx_v)
            pltpu.sync_copy(mref.at[pl.ds(base, _CH)], mbuf)
            pltpu.sync_copy(mbuf, acc.at[idx_v], add=True)
        _chunk_loop(s, one)

    def wb(out):
        def one(base):
            pltpu.sync_copy(nid.at[pl.ds(base, _CH)], idx_v)
            pltpu.sync_copy(acc.at[idx_v], mbuf)
            pltpu.sync_copy(mbuf, out.at[pl.ds(base, _CH)])
        _rows_loop(one)

    zero_acc()
    plsc.subcore_barrier()

    @pl.when(c == 0)
    def _():
        madd(m0)

    @pl.when(c == 1)
    def _():
        madd(m1)

    plsc.subcore_barrier()

    @pl.when(c == 0)
    def _():
        wb(agg0)

    @pl.when(c == 1)
    def _():
        wb(agg1)

    plsc.subcore_barrier()
    zero_acc()
    plsc.subcore_barrier()

    nk2 = (_NCHUNK // 2 - s + _NS - 1) // _NS

    def step2(i, carry):
        base = (c + 2 * (s + i * _NS)) * _CH
        pltpu.sync_copy(row.at[pl.ds(base, _CH)], idx_v)
        pltpu.sync_copy(rc.at[pl.ds(base, _CH)], mbuf)
        pltpu.sync_copy(mbuf, acc.at[idx_v], add=True)
        return carry
    jax.lax.fori_loop(0, nk2, step2, 0)

    plsc.subcore_barrier()

    @pl.when(c == 0)
    def _():
        wb(aggp0)

    @pl.when(c == 1)
    def _():
        wb(aggp1)


def _sc_scatter(m0, m1, rc, row, nid, z128):
    if 'sc' not in _sc_cache:
        _sc_cache['sc'] = pl.kernel(
            _sc_scatter_body,
            out_type=[_f32(_NPAD, H // 2), _f32(_NPAD, H // 2),
                      _f32(_NPAD, 128), _f32(_NPAD, 128)],
            mesh=_sc_mesh(),
            scratch_types=[pltpu.VMEM((_CH,), jnp.int32),
                           pltpu.VMEM((_CH, H // 2), jnp.float32),
                           pltpu.VMEM_SHARED((_NPAD, H // 2), jnp.float32)],
        )
    return _sc_cache['sc'](m0, m1, rc, row, nid, z128)


def _temb_body(ts_ref, w0, b0, w1, b1, w2, b2, out_ref):
    ts = ts_ref[...].astype(jnp.float32)
    j = jax.lax.broadcasted_iota(jnp.int32, (1, H // 2), 1).astype(jnp.float32)
    freqs = jnp.exp(-np.log(10000.0) * j / (H // 2 - 1))
    args = ts * freqs
    te = jnp.concatenate([jnp.sin(args), jnp.cos(args)], axis=1)
    x = _mm(te, w0[...]) + b0[...]
    x = _relu(x)
    x = _mm(x, w1[...]) + b1[...]
    x = _mm(_relu(x), w2[...]) + b2[...]
    out_ref[...] = x


def _temb(time_step, p):
    return pl.pallas_call(
        _temb_body,
        out_shape=_f32(G, H),
    )(time_step.reshape(G, 1),
      p['temb0_w'], p['temb0_b'].reshape(1, -1),
      p['temb1_w'], p['temb1_b'].reshape(1, -1),
      p['temb_proj_w'], p['temb_proj_b'].reshape(1, -1))


def _nodepro_body(at_ref, b_ref, temb_ref, ea_w, eb_w, e_b, h_ref, tn_ref):
    oh = (b_ref[...] == jax.lax.broadcasted_iota(jnp.int32, (TN, G), 1))
    tn = _mm(oh.astype(jnp.float32), temb_ref[...])
    h_ref[...] = _mm(at_ref[...], ea_w[...]) + _mm(tn, eb_w[...]) + e_b[...]
    tn_ref[...] = tn


def _node_prologue(atom_type, batch, temb, p):
    emb_a = p['emb_in_w'][:A]
    emb_b = p['emb_in_w'][A:]
    return pl.pallas_call(
        _nodepro_body,
        grid=(N // TN,),
        in_specs=[_rows(TN, A), _rows(TN, 1), _fixed((G, H)),
                  _fixed((A, H)), _fixed((H, H)), _fixed((1, H))],
        out_specs=[_rows(TN, H), _rows(TN, H)],
        out_shape=[_f32(N, H), _f32(N, H)],
    )(atom_type, batch.reshape(N, 1).astype(jnp.int32), temb,
      emb_a, emb_b, p['emb_in_b'].reshape(1, H))


def _edgepro_body(el_ref, et_ref, ee1w, ee1b, ee2w, ee2b, bond4, eab_ref):
    ea = _relu(el_ref[...] * ee1w[...] + ee1b[...])
    ea = _mm(ea, ee2w[...]) + ee2b[...]
    oh = (et_ref[...] == jax.lax.broadcasted_iota(jnp.int32, (TE, 4), 1))
    eab_ref[...] = ea * _mm(oh.astype(jnp.float32), bond4[...])


def _edge_prologue(edge_length, edge_type, p):
    return pl.pallas_call(
        _edgepro_body,
        grid=(E // TE,),
        in_specs=[_rows(TE, 1), _rows(TE, 1), _fixed((1, H)), _fixed((1, H)),
                  _fixed((H, H)), _fixed((1, H)), _fixed((4, H))],
        out_specs=_rows(TE, H),
        out_shape=_f32(E, H),
    )(edge_length, edge_type.reshape(E, 1).astype(jnp.int32),
      p['ee1_w'], p['ee1_b'].reshape(1, H),
      p['ee2_w'], p['ee2_b'].reshape(1, H), p['bond_emb'][:4])


def _nodeside_body(h_ref, tn_ref, pp_ref, wrow, wcol, wea, rt_ref, ct_ref):
    h = h_ref[...]
    pp = pp_ref[...]
    rt_ref[...] = jnp.concatenate(
        [_mm(h, wrow[...]) + _mm(tn_ref[...], wea[...]), pp], axis=1)
    ct_ref[...] = jnp.concatenate([_mm(h, wcol[...]), pp], axis=1)


def _nodeside(h, tn, pp, wrow, wcol, wea):
    return pl.pallas_call(
        _nodeside_body,
        grid=(N // TN,),
        in_specs=[_rows(TN, H), _rows(TN, H), _rows(TN, 128),
                  _fixed((H, H)), _fixed((H, H)), _fixed((H, H))],
        out_specs=[_rows(TN, H + 128), _rows(TN, H + 128)],
        out_shape=[_f32(N, H + 128), _f32(N, H + 128)],
    )(h, tn, pp, wrow, wcol, wea)


def _edge_body(grow, gcol, eab, wea, wd2, b1, w2, b2,
               wsoft, bsoft, wc1, bc1, wc2, bc2, m0_ref, m1_ref, rc_ref):
    gr = grow[...]
    gc = gcol[...]
    rel = gr[:, H:] - gc[:, H:]
    d2 = jnp.sum(rel * rel, axis=1, keepdims=True)
    reln = rel / (jnp.sqrt(d2) + 1e-8)
    m1 = _silu(gr[:, :H] + gc[:, :H] + _mm(eab[...], wea[...])
               + d2 * wd2[...] + b1[...])
    m = _silu(_mm(m1, w2[...]) + b2[...])
    gate = _sigmoid(_mm(m, wsoft[...]) + bsoft[...])
    m = m * gate
    cw = _silu(_mm(m, wc1[...]) + bc1[...])
    cws = _mm(cw, wc2[...]) + bc2[...]
    m0_ref[...] = m[:, :H // 2]
    m1_ref[...] = m[:, H // 2:]
    rc_ref[...] = reln * cws


def _edge_layer(grow, gcol, eab, wea, wd2, b1, w2, b2,
                wsoft, bsoft, wc1, bc1, wc2, bc2):
    return pl.pallas_call(
        _edge_body,
        grid=(E // TE,),
        in_specs=[_rows(TE, H + 128), _rows(TE, H + 128),
                  _rows(TE, H),
                  _fixed((H, H)), _fixed((1, H)), _fixed((1, H)),
                  _fixed((H, H)), _fixed((1, H)),
                  _fixed((H, 1)), _fixed((1, 1)),
                  _fixed((H, H)), _fixed((1, H)),
                  _fixed((H, 1)), _fixed((1, 1))],
        out_specs=[_rows(TE, H // 2), _rows(TE, H // 2), _rows(TE, 128)],
        out_shape=[_f32(E, H // 2), _f32(E, H // 2), _f32(E, 128)],
    )(grow, gcol, eab, wea, wd2, b1, w2, b2,
      wsoft, bsoft, wc1, bc1, wc2, bc2)


def _nodeupd_body(h_ref, agg0_ref, agg1_ref, aggp0_ref, aggp1_ref, p_ref,
                  n1a, n1b0, n1b1, nb1, n2w, n2b, ho_ref, po_ref):
    h = h_ref[...]
    upd = _silu(_mm(h, n1a[...]) + _mm(agg0_ref[...], n1b0[...])
                + _mm(agg1_ref[...], n1b1[...]) + nb1[...])
    ho_ref[...] = h + _mm(upd, n2w[...]) + n2b[...]
    po_ref[...] = p_ref[...] + aggp0_ref[...] + aggp1_ref[...]


def _node_update(h, agg0, agg1, aggp0, aggp1, pp, n1a, n1b0, n1b1, nb1, n2w, n2b):
    return pl.pallas_call(
        _nodeupd_body,
        grid=(N // TN,),
        in_specs=[_rows(TN, H), _rows(TN, H // 2), _rows(TN, H // 2),
                  _rows(TN, 128), _rows(TN, 128), _rows(TN, 128),
                  _fixed((H, H)), _fixed((H // 2, H)), _fixed((H // 2, H)),
                  _fixed((1, H)), _fixed((H, H)), _fixed((1, H))],
        out_specs=[_rows(TN, H), _rows(TN, 128)],
        out_shape=[_f32(N, H), _f32(N, 128)],
    )(h, agg0, agg1, aggp0, aggp1, pp, n1a, n1b0, n1b1, nb1, n2w, n2b)


def _nodehead_body(h_ref, tn_ref, g1w, g1b, g2w, g2b, g3w, g3b, wde,
                   ns_ref, tf_ref):
    h = h_ref[...]
    g = _relu(_mm(h, g1w[...]) + g1b[...])
    g = _relu(_mm(g, g2w[...]) + g2b[...])
    ns_ref[...] = _mm(g, g3w[...]) + g3b[...]
    tf_ref[...] = jnp.concatenate([h, _mm(tn_ref[...], wde[...])], axis=1)


def _node_head(h, tn, p, wde):
    return pl.pallas_call(
        _nodehead_body,
        grid=(N // TN,),
        in_specs=[_rows(TN, H), _rows(TN, H),
                  _fixed((H, H)), _fixed((1, H)),
                  _fixed((H, H // 2)), _fixed((1, H // 2)),
                  _fixed((H // 2, A)), _fixed((1, A)), _fixed((H, H))],
        out_specs=[_rows(TN, A), _rows(TN, 2 * H)],
        out_shape=[_f32(N, A), _f32(N, 2 * H)],
    )(h, tn, p['g1_w'], p['g1_b'].reshape(1, -1),
      p['g2_w'], p['g2_b'].reshape(1, -1),
      p['g3_w'], p['g3_b'].reshape(1, -1), wde)


def _disthead_body(hr, hc, tfr, eab, wdh, wde, d1b, d2w, d2b, d3w, d3b, out):
    hp = _relu(_mm(hr[...] * hc[...], wdh[...]) + _mm(eab[...], wde[...])
               + tfr[...] + d1b[...])
    d = _relu(_mm(hp, d2w[...]) + d2b[...])
    out[...] = _mm(d, d3w[...]) + d3b[...]


def _dist_head(growf, gcolf, eab, p):
    wdh = p['d1_w'][:H]
    wde = p['d1_w'][H:]
    hr_spec = pl.BlockSpec((TE, H), lambda i: (i, 0))
    tfr_spec = pl.BlockSpec((TE, H), lambda i: (i, 1))
    return pl.pallas_call(
        _disthead_body,
        grid=(E // TE,),
        in_specs=[hr_spec, _rows(TE, H), tfr_spec, _rows(TE, H),
                  _fixed((H, H)), _fixed((H, H)), _fixed((1, H)),
                  _fixed((H, H // 2)), _fixed((1, H // 2)),
                  _fixed((H // 2, 1)), _fixed((1, 1))],
        out_specs=_rows(TE, 1),
        out_shape=_f32(E, 1),
    )(growf, gcolf, growf, eab, wdh, wde, p['d1_b'].reshape(1, -1),
      p['d2_w'], p['d2_b'].reshape(1, -1), p['d3_w'], p['d3_b'].reshape(1, -1))


def kernel(atom_type, pos, bond_index, batch, time_step, edge_type, edge_index, edge_length, params):
    p = params
    row = edge_index[0]
    col = edge_index[1]

    temb = _temb(time_step, p)
    h, tn = _node_prologue(atom_type, batch, temb, p)
    eab = _edge_prologue(edge_length, edge_type, p)

    pp = jnp.concatenate([pos, jnp.zeros((N, 125), jnp.float32)], axis=1)
    nid = jnp.arange(_NPAD, dtype=jnp.int32)
    z128 = jnp.zeros((_NPAD, H // 2), jnp.float32)

    for l in range(L):
        w1 = p[f'e{l}_1_w']
        rt, ct = _nodeside(h, tn, pp, w1[:H], w1[H:2 * H], w1[2 * H + 1:])
        grow, gcol = _sc_gather_layer(rt, ct, row, col)
        m0, m1, rc = _edge_layer(
            grow, gcol, eab,
            w1[2 * H + 1:], w1[2 * H:2 * H + 1], p[f'e{l}_1_b'].reshape(1, H),
            p[f'e{l}_2_w'], p[f'e{l}_2_b'].reshape(1, H),
            p[f'e{l}_soft_w'], p[f'e{l}_soft_b'].reshape(1, 1),
            p[f'c{l}_1_w'], p[f'c{l}_1_b'].reshape(1, H),
            p[f'c{l}_2_w'], p[f'c{l}_2_b'].reshape(1, 1))
        agg0, agg1, aggp0, aggp1 = _sc_scatter(m0, m1, rc, row, nid, z128)
        n1w = p[f'n{l}_1_w']
        h, pp = _node_update(h, agg0, agg1, aggp0, aggp1, pp,
                             n1w[:H], n1w[H:H + H // 2], n1w[H + H // 2:],
                             p[f'n{l}_1_b'].reshape(1, H),
                             p[f'n{l}_2_w'], p[f'n{l}_2_b'].reshape(1, H))

    node_score, tabf = _node_head(h, tn, p, p['d1_w'][H:])
    growf, gcolf = _sc_gather_final(tabf, h, row, col)
    dist_score = _dist_head(growf, gcolf, eab, p)

    local_edge_mask = (edge_type == 0)
    return (dist_score, node_score, edge_index, edge_type, edge_length, local_edge_mask)
